# Initial kernel scaffold; baseline (speedup 1.0000x reference)
import jax
import jax.numpy as jnp
from jax.experimental import pallas as pl


def kernel(x, edge_index, Wz, bz, Wr, br, Wh, bh, lzW, lzb, lrW, lrb, lhW, lhb, att, linW, linb):
    raise NotImplementedError("write your pallas kernel here")


# speedup vs baseline: 95.1073x; 95.1073x over previous
import jax
import jax.numpy as jnp
from jax.experimental import pallas as pl
from jax.experimental.pallas import tpu as pltpu

_N = 207
_F = 2
_P = 12
_B = 12
_MID = _N * 5
_E = 1722
_BP = _B * _P

_NP = 256
_EP = 2048
_MP = 1152

_PREC = jax.lax.Precision.HIGHEST


def _prep_body(row_ref, col_ref, x0_ref, x1_ref, wz_ref, wh_ref, lz_ref,
               lh_ref, bz_ref, lzb_ref, bh_ref, lhb_ref,
               y0_ref, y1_ref, cz_ref, ch_ref, dz_ref, dh_ref):
    row = row_ref[...]
    col = col_ref[...]
    ids = jax.lax.broadcasted_iota(jnp.int32, (_NP, _EP), 0)
    oh_row = (ids == row).astype(jnp.float32)
    oh_col = (ids == col).astype(jnp.float32)
    deg = jnp.sum(oh_col, axis=1, keepdims=True)
    dis = jnp.where(deg > 0.0, jax.lax.rsqrt(deg), 0.0)
    dis_row = jnp.sum(oh_row * dis, axis=0, keepdims=True)
    dis_col = jnp.sum(oh_col * dis, axis=0, keepdims=True)
    normv = dis_row * dis_col
    st = jax.lax.dot_general(oh_row * normv, oh_col,
                             (((1,), (1,)), ((), ())),
                             preferred_element_type=jnp.float32,
                             precision=_PREC)
    y0_ref[...] = jax.lax.dot_general(x0_ref[...], st,
                                      (((1,), (0,)), ((), ())),
                                      preferred_element_type=jnp.float32,
                                      precision=_PREC)
    y1_ref[...] = jax.lax.dot_general(x1_ref[...], st,
                                      (((1,), (0,)), ((), ())),
                                      preferred_element_type=jnp.float32,
                                      precision=_PREC)
    cz_ref[...] = jax.lax.dot_general(wz_ref[...], lz_ref[...],
                                      (((0,), (1,)), ((), ())),
                                      preferred_element_type=jnp.float32,
                                      precision=_PREC)
    ch_ref[...] = jax.lax.dot_general(wh_ref[...], lh_ref[...],
                                      (((0,), (1,)), ((), ())),
                                      preferred_element_type=jnp.float32,
                                      precision=_PREC)
    dz_ref[...] = jax.lax.dot_general(bz_ref[...], lz_ref[...],
                                      (((1,), (1,)), ((), ())),
                                      preferred_element_type=jnp.float32,
                                      precision=_PREC) + lzb_ref[...]
    dh_ref[...] = jax.lax.dot_general(bh_ref[...], lh_ref[...],
                                      (((1,), (1,)), ((), ())),
                                      preferred_element_type=jnp.float32,
                                      precision=_PREC) + lhb_ref[...]


def _gate_body(y0_ref, y1_ref, cz_ref, ch_ref, dz_ref, dh_ref, att_ref,
               lin_ref, linb_ref, out_ref, acc_ref):
    p = pl.program_id(1)
    y0 = y0_ref[0]
    y1 = y1_ref[0]
    pz = y0 * cz_ref[0:1, :] + y1 * cz_ref[1:2, :] + dz_ref[...]
    ph = y0 * ch_ref[0:1, :] + y1 * ch_ref[1:2, :] + dh_ref[...]
    hp = (1.0 - jax.nn.sigmoid(pz)) * jnp.tanh(ph)
    att = att_ref[...]
    e = jnp.exp(att - jnp.max(att))
    w_all = e / jnp.sum(e)
    lane = jax.lax.broadcasted_iota(jnp.int32, (1, _P), 1)
    w = jnp.sum(jnp.where(lane == p, w_all, 0.0))
    contrib = w * hp

    @pl.when(p == 0)
    def _init():
        acc_ref[...] = contrib

    @pl.when(p > 0)
    def _accum():
        acc_ref[...] = acc_ref[...] + contrib

    @pl.when(p == _P - 1)
    def _head():
        h = jnp.maximum(acc_ref[...], 0.0)
        o = jax.lax.dot_general(h, lin_ref[...],
                                (((1,), (1,)), ((), ())),
                                preferred_element_type=jnp.float32,
                                precision=_PREC)
        out_ref[0] = jax.nn.sigmoid(o + linb_ref[...])


def kernel(x, edge_index, Wz, bz, Wr, br, Wh, bh, lzW, lzb, lrW, lrb,
           lhW, lhb, att, linW, linb):
    f32 = jnp.float32
    loop = jnp.arange(_N, dtype=edge_index.dtype)
    row = jnp.concatenate([edge_index[0], loop])
    col = jnp.concatenate([edge_index[1], loop])
    pad_e = _EP - row.shape[0]
    row = jnp.pad(row, (0, pad_e), constant_values=_N).astype(jnp.int32)
    col = jnp.pad(col, (0, pad_e), constant_values=_N).astype(jnp.int32)
    row2 = row.reshape(1, _EP)
    col2 = col.reshape(1, _EP)

    xt = x.transpose(0, 3, 2, 1)
    x0t = jnp.pad(xt[:, :, 0, :].reshape(_BP, _N), ((0, 0), (0, _NP - _N)))
    x1t = jnp.pad(xt[:, :, 1, :].reshape(_BP, _N), ((0, 0), (0, _NP - _N)))

    lz = jnp.pad(lzW[:, :_MID], ((0, _MP - _MID), (0, _MP - _MID)))
    lh = jnp.pad(lhW[:, :_MID], ((0, _MP - _MID), (0, _MP - _MID)))
    wz = jnp.pad(Wz, ((0, _MP - _MID), (0, 0)))
    wh = jnp.pad(Wh, ((0, _MP - _MID), (0, 0)))
    bz2 = jnp.pad(bz, (0, _MP - _MID)).reshape(1, _MP)
    bh2 = jnp.pad(bh, (0, _MP - _MID)).reshape(1, _MP)
    lzb2 = jnp.pad(lzb, (0, _MP - _MID)).reshape(1, _MP)
    lhb2 = jnp.pad(lhb, (0, _MP - _MID)).reshape(1, _MP)
    lin = jnp.pad(linW, ((0, 0), (0, _MP - _MID)))
    linb2 = linb.reshape(1, _P)
    att2 = att.reshape(1, _P)

    y0t, y1t, cz, ch, dz, dh = pl.pallas_call(
        _prep_body,
        out_shape=(
            jax.ShapeDtypeStruct((_BP, _NP), f32),
            jax.ShapeDtypeStruct((_BP, _NP), f32),
            jax.ShapeDtypeStruct((_F, _MP), f32),
            jax.ShapeDtypeStruct((_F, _MP), f32),
            jax.ShapeDtypeStruct((1, _MP), f32),
            jax.ShapeDtypeStruct((1, _MP), f32),
        ),
    )(row2, col2, x0t, x1t, wz, wh, lz, lh, bz2, lzb2, bh2, lhb2)

    y0r = y0t.reshape(_BP, _NP, 1)
    y1r = y1t.reshape(_BP, _NP, 1)

    full = lambda s: pl.BlockSpec(s, lambda b, p: tuple(0 for _ in s))
    out3 = pl.pallas_call(
        _gate_body,
        grid=(_B, _P),
        in_specs=[
            pl.BlockSpec((1, _NP, 1), lambda b, p: (b * _P + p, 0, 0)),
            pl.BlockSpec((1, _NP, 1), lambda b, p: (b * _P + p, 0, 0)),
            full((_F, _MP)),
            full((_F, _MP)),
            full((1, _MP)),
            full((1, _MP)),
            full((1, _P)),
            full((_P, _MP)),
            full((1, _P)),
        ],
        out_specs=pl.BlockSpec((1, _NP, _P), lambda b, p: (b, 0, 0)),
        out_shape=jax.ShapeDtypeStruct((_B, _NP, _P), f32),
        scratch_shapes=[pltpu.VMEM((_NP, _MP), f32)],
        compiler_params=pltpu.CompilerParams(
            dimension_semantics=("arbitrary", "arbitrary")),
    )(y0r, y1r, cz, ch, dz, dh, att2, lin, linb2)

    return out3[:, :_N, :]
